# Initial kernel scaffold; baseline (speedup 1.0000x reference)
import jax
import jax.numpy as jnp
from jax.experimental import pallas as pl


def kernel(x, table, W, b):
    raise NotImplementedError("write your pallas kernel here")


# speedup vs baseline: 34.8433x; 34.8433x over previous
import functools

import jax
import jax.numpy as jnp
from jax import lax
from jax.experimental import pallas as pl
from jax.experimental.pallas import tpu as pltpu
from jax.experimental.pallas import tpu_sc as plsc

_VOCAB = 1001
_DIM = 64
_B = 16384
_L = 200
_VPAD = 1024

_NC, _NS, _LANES = 2, 16, 16
_NW = _NC * _NS
_GRP = _LANES
_NGROUPS = _B // _GRP
_GPW = _NGROUPS // _NW


def _count_body(x_hbm, c_hbm, x_v, c_v):
    wid = lax.axis_index("s") * _NC + lax.axis_index("c")
    lane = lax.iota(jnp.int32, _LANES)
    col_base = lane * _L
    row_base = lane * _VPAD
    ones = jnp.full((_LANES,), 1.0, jnp.float32)
    neg_ones = jnp.full((_LANES,), -1.0, jnp.float32)

    def _zero(j, _):
        c_v[pl.ds(j * _LANES, _LANES)] = jnp.zeros((_LANES,), jnp.float32)
        return _
    lax.fori_loop(0, (_GRP * _VPAD) // _LANES, _zero, None)

    def _scatter_pass(val):
        def _step(l, _):
            xv = plsc.load_gather(x_v, [col_base + l])
            plsc.addupdate_scatter(c_v, [row_base + xv], val)
            return _
        lax.fori_loop(0, _L, _step, None)

    def _group(g_local, _):
        g = wid * _GPW + g_local
        pltpu.sync_copy(x_hbm.at[g], x_v)
        _scatter_pass(ones)
        pltpu.sync_copy(c_v, c_hbm.at[g])
        _scatter_pass(neg_ones)
        return _
    lax.fori_loop(0, _GPW, _group, None)


@functools.cache
def _make_count():
    return pl.kernel(
        _count_body,
        out_type=jax.ShapeDtypeStruct((_NGROUPS, _GRP * _VPAD), jnp.float32),
        mesh=plsc.VectorSubcoreMesh(core_axis_name="c", subcore_axis_name="s"),
        scratch_types=[
            pltpu.VMEM((_GRP * _L,), jnp.int32),
            pltpu.VMEM((_GRP * _VPAD,), jnp.float32),
        ],
        compiler_params=pltpu.CompilerParams(needs_layout_passes=False),
    )


_BLK = 1024


def _proj_body(c_ref, t_ref, w_ref, b_ref, o_ref):
    pooled = jnp.dot(c_ref[...], t_ref[...],
                     preferred_element_type=jnp.float32) * (1.0 / _L)
    o_ref[...] = jnp.dot(pooled, w_ref[...],
                         preferred_element_type=jnp.float32) + b_ref[...]


_proj = pl.pallas_call(
    _proj_body,
    grid=(_B // _BLK,),
    in_specs=[
        pl.BlockSpec((_BLK, _VPAD), lambda i: (i, 0)),
        pl.BlockSpec((_VPAD, _DIM), lambda i: (0, 0)),
        pl.BlockSpec((_DIM, _VOCAB), lambda i: (0, 0)),
        pl.BlockSpec((1, _VOCAB), lambda i: (0, 0)),
    ],
    out_specs=pl.BlockSpec((_BLK, _VOCAB), lambda i: (i, 0)),
    out_shape=jax.ShapeDtypeStruct((_B, _VOCAB), jnp.float32),
    compiler_params=pltpu.CompilerParams(
        dimension_semantics=("arbitrary",)),
)


def kernel(x, table, W, b):
    xr = x.astype(jnp.int32).reshape(_NGROUPS, _GRP * _L)
    C = _make_count()(xr).reshape(_B, _VPAD)
    table_p = jnp.pad(table, ((0, _VPAD - _VOCAB), (0, 0)))
    return _proj(C, table_p, W.T, b.reshape(1, _VOCAB))


# speedup vs baseline: 45.0839x; 1.2939x over previous
import functools

import jax
import jax.numpy as jnp
from jax import lax
from jax.experimental import pallas as pl
from jax.experimental.pallas import tpu as pltpu
from jax.experimental.pallas import tpu_sc as plsc

_VOCAB = 1001
_DIM = 64
_B = 16384
_L = 200
_VPAD = 1024

_NC, _NS, _LANES = 2, 16, 16
_NW = _NC * _NS
_GRP = _LANES
_NGROUPS = _B // _GRP
_GPW = _NGROUPS // _NW


def _count_body(x_hbm, c_hbm, x_v, c_v):
    wid = lax.axis_index("s") * _NC + lax.axis_index("c")
    lane = lax.iota(jnp.int32, _LANES)
    col_base = lane * _L
    row_base = lane * _VPAD
    ones = jnp.full((_LANES,), 1.0, jnp.float32)
    neg_ones = jnp.full((_LANES,), -1.0, jnp.float32)

    @plsc.parallel_loop(0, _GRP * _VPAD, step=_LANES, unroll=8)
    def _zero(j):
        c_v[pl.ds(j, _LANES)] = jnp.zeros((_LANES,), jnp.float32)

    def _scatter_pass(val):
        @plsc.parallel_loop(0, _L, unroll=8)
        def _step(l):
            xv = plsc.load_gather(x_v, [col_base + l])
            plsc.addupdate_scatter(c_v, [row_base + xv], val)

    def _group(g_local, _):
        g = wid * _GPW + g_local
        pltpu.sync_copy(x_hbm.at[g], x_v)
        _scatter_pass(ones)
        pltpu.sync_copy(c_v, c_hbm.at[g])
        _scatter_pass(neg_ones)
        return _
    lax.fori_loop(0, _GPW, _group, None)


@functools.cache
def _make_count():
    return pl.kernel(
        _count_body,
        out_type=jax.ShapeDtypeStruct((_NGROUPS, _GRP * _VPAD), jnp.float32),
        mesh=plsc.VectorSubcoreMesh(core_axis_name="c", subcore_axis_name="s"),
        scratch_types=[
            pltpu.VMEM((_GRP * _L,), jnp.int32),
            pltpu.VMEM((_GRP * _VPAD,), jnp.float32),
        ],
        compiler_params=pltpu.CompilerParams(needs_layout_passes=False),
    )


_BLK = 1024


def _proj_body(c_ref, t_ref, w_ref, b_ref, o_ref):
    pooled = jnp.dot(c_ref[...], t_ref[...],
                     preferred_element_type=jnp.float32) * (1.0 / _L)
    o_ref[...] = jnp.dot(pooled, w_ref[...],
                         preferred_element_type=jnp.float32) + b_ref[...]


_proj = pl.pallas_call(
    _proj_body,
    grid=(_B // _BLK,),
    in_specs=[
        pl.BlockSpec((_BLK, _VPAD), lambda i: (i, 0)),
        pl.BlockSpec((_VPAD, _DIM), lambda i: (0, 0)),
        pl.BlockSpec((_DIM, _VOCAB), lambda i: (0, 0)),
        pl.BlockSpec((1, _VOCAB), lambda i: (0, 0)),
    ],
    out_specs=pl.BlockSpec((_BLK, _VOCAB), lambda i: (i, 0)),
    out_shape=jax.ShapeDtypeStruct((_B, _VOCAB), jnp.float32),
    compiler_params=pltpu.CompilerParams(
        dimension_semantics=("arbitrary",)),
)


def kernel(x, table, W, b):
    xr = x.astype(jnp.int32).reshape(_NGROUPS, _GRP * _L)
    C = _make_count()(xr).reshape(_B, _VPAD)
    table_p = jnp.pad(table, ((0, _VPAD - _VOCAB), (0, 0)))
    return _proj(C, table_p, W.T, b.reshape(1, _VOCAB))


# speedup vs baseline: 47.4738x; 1.0530x over previous
import functools

import jax
import jax.numpy as jnp
from jax import lax
from jax.experimental import pallas as pl
from jax.experimental.pallas import tpu as pltpu
from jax.experimental.pallas import tpu_sc as plsc

_VOCAB = 1001
_DIM = 64
_B = 16384
_L = 200
_VPAD = 1024

_NC, _NS, _LANES = 2, 16, 16
_NW = _NC * _NS
_GRP = _LANES
_NGROUPS = _B // _GRP
_GPW = _NGROUPS // _NW


def _count_body(x_hbm, c_hbm, x_v, c_v):
    wid = lax.axis_index("s") * _NC + lax.axis_index("c")
    lane = lax.iota(jnp.int32, _LANES)
    ones = jnp.full((_LANES,), 1.0, jnp.float32)
    neg_ones = jnp.full((_LANES,), -1.0, jnp.float32)

    for r in range(_GRP):
        @plsc.parallel_loop(0, _VPAD, step=_LANES, unroll=8)
        def _zero(j, r=r):
            c_v[r, pl.ds(j, _LANES)] = jnp.zeros((_LANES,), jnp.float32)

    def _scatter_pass(val):
        @plsc.parallel_loop(0, _L, unroll=8)
        def _step(l):
            xv = plsc.load_gather(x_v, [lane, jnp.full((_LANES,), l)])
            plsc.addupdate_scatter(c_v, [lane, xv], val)

    def _group(g_local, _):
        g = wid * _GPW + g_local
        base = g * _GRP
        pltpu.sync_copy(x_hbm.at[pl.ds(base, _GRP), :], x_v)
        _scatter_pass(ones)
        pltpu.sync_copy(c_v, c_hbm.at[pl.ds(base, _GRP), :])
        _scatter_pass(neg_ones)
        return _
    lax.fori_loop(0, _GPW, _group, None)


@functools.cache
def _make_count():
    return pl.kernel(
        _count_body,
        out_type=jax.ShapeDtypeStruct((_B, _VPAD), jnp.float32),
        mesh=plsc.VectorSubcoreMesh(core_axis_name="c", subcore_axis_name="s"),
        scratch_types=[
            pltpu.VMEM((_GRP, _L), jnp.int32),
            pltpu.VMEM((_GRP, _VPAD), jnp.float32),
        ],
        compiler_params=pltpu.CompilerParams(
            needs_layout_passes=False, use_tc_tiling_on_sc=True),
    )


_BLK = 1024


def _proj_body(c_ref, t_ref, w_ref, b_ref, o_ref):
    pooled = jnp.dot(c_ref[...], t_ref[...],
                     preferred_element_type=jnp.float32) * (1.0 / _L)
    o_ref[...] = jnp.dot(pooled, w_ref[...],
                         preferred_element_type=jnp.float32) + b_ref[...]


_proj = pl.pallas_call(
    _proj_body,
    grid=(_B // _BLK,),
    in_specs=[
        pl.BlockSpec((_BLK, _VPAD), lambda i: (i, 0)),
        pl.BlockSpec((_VPAD, _DIM), lambda i: (0, 0)),
        pl.BlockSpec((_DIM, _VOCAB), lambda i: (0, 0)),
        pl.BlockSpec((1, _VOCAB), lambda i: (0, 0)),
    ],
    out_specs=pl.BlockSpec((_BLK, _VOCAB), lambda i: (i, 0)),
    out_shape=jax.ShapeDtypeStruct((_B, _VOCAB), jnp.float32),
    compiler_params=pltpu.CompilerParams(
        dimension_semantics=("arbitrary",)),
)


def kernel(x, table, W, b):
    C = _make_count()(x.astype(jnp.int32))
    table_p = jnp.pad(table, ((0, _VPAD - _VOCAB), (0, 0)))
    return _proj(C, table_p, W.T, b.reshape(1, _VOCAB))


# speedup vs baseline: 56.6793x; 1.1939x over previous
import functools

import jax
import jax.numpy as jnp
from jax import lax
from jax.experimental import pallas as pl
from jax.experimental.pallas import tpu as pltpu
from jax.experimental.pallas import tpu_sc as plsc

_VOCAB = 1001
_DIM = 64
_B = 16384
_L = 200
_VPAD = 1024

_NC, _NS, _LANES = 2, 16, 16
_NW = _NC * _NS
_GRP = _LANES
_NGROUPS = _B // _GRP
_GPW = _NGROUPS // _NW


def _count_body(x_hbm, c_hbm, x_v, c_v):
    wid = lax.axis_index("s") * _NC + lax.axis_index("c")
    lane = lax.iota(jnp.int32, _LANES)
    ones = jnp.full((_LANES,), 1.0, jnp.float32)
    neg_ones = jnp.full((_LANES,), -1.0, jnp.float32)

    for r in range(_GRP):
        @plsc.parallel_loop(0, _VPAD, step=_LANES, unroll=8)
        def _zero(j, r=r):
            c_v[r, pl.ds(j, _LANES)] = jnp.zeros((_LANES,), jnp.float32)

    def _scatter_pass(val):
        @plsc.parallel_loop(0, _L, unroll=8)
        def _step(l):
            xv = plsc.load_gather(x_v, [lane, jnp.full((_LANES,), l)])
            plsc.addupdate_scatter(c_v, [lane, xv], val)

    def _group(g_local, _):
        g = wid * _GPW + g_local
        base = g * _GRP
        pltpu.sync_copy(x_hbm.at[pl.ds(base, _GRP), :], x_v)
        _scatter_pass(ones)
        pltpu.sync_copy(c_v, c_hbm.at[pl.ds(base, _GRP), :])
        _scatter_pass(neg_ones)
        return _
    lax.fori_loop(0, _GPW, _group, None)


@functools.cache
def _make_count():
    return pl.kernel(
        _count_body,
        out_type=jax.ShapeDtypeStruct((_B, _VPAD), jnp.float32),
        mesh=plsc.VectorSubcoreMesh(core_axis_name="c", subcore_axis_name="s"),
        scratch_types=[
            pltpu.VMEM((_GRP, _L), jnp.int32),
            pltpu.VMEM((_GRP, _VPAD), jnp.float32),
        ],
        compiler_params=pltpu.CompilerParams(
            needs_layout_passes=False, use_tc_tiling_on_sc=False),
    )


_BLK = 1024


def _proj_body(c_ref, t_ref, w_ref, b_ref, o_ref):
    pooled = jnp.dot(c_ref[...], t_ref[...],
                     preferred_element_type=jnp.float32) * (1.0 / _L)
    logits_t = lax.dot_general(
        w_ref[...], pooled, (((1,), (1,)), ((), ())),
        preferred_element_type=jnp.float32)
    o_ref[...] = logits_t + b_ref[...]


_proj = pl.pallas_call(
    _proj_body,
    grid=(_B // _BLK,),
    in_specs=[
        pl.BlockSpec((_BLK, _VPAD), lambda i: (i, 0)),
        pl.BlockSpec((_VPAD, _DIM), lambda i: (0, 0)),
        pl.BlockSpec((_VOCAB, _DIM), lambda i: (0, 0)),
        pl.BlockSpec((_VOCAB, 1), lambda i: (0, 0)),
    ],
    out_specs=pl.BlockSpec((_VOCAB, _BLK), lambda i: (0, i)),
    out_shape=jax.ShapeDtypeStruct((_VOCAB, _B), jnp.float32),
    compiler_params=pltpu.CompilerParams(
        dimension_semantics=("arbitrary",)),
)


def kernel(x, table, W, b):
    C = _make_count()(x.astype(jnp.int32))
    table_p = jnp.pad(table, ((0, _VPAD - _VOCAB), (0, 0)))
    return _proj(C, table_p, W, b.reshape(_VOCAB, 1)).T


# speedup vs baseline: 73.8165x; 1.3024x over previous
import functools

import jax
import jax.numpy as jnp
from jax import lax
from jax.experimental import pallas as pl
from jax.experimental.pallas import tpu as pltpu
from jax.experimental.pallas import tpu_sc as plsc

_VOCAB = 1001
_DIM = 64
_B = 16384
_L = 200
_VPAD = 1024

_NC, _NS, _LANES = 2, 16, 16
_NW = _NC * _NS
_GRP = _LANES
_NGROUPS = _B // _GRP
_GPW = _NGROUPS // _NW


def _count_body(x_hbm, c_hbm, x_v, c_v):
    wid = lax.axis_index("s") * _NC + lax.axis_index("c")
    lane = lax.iota(jnp.int32, _LANES)
    ones = jnp.full((_LANES,), 1.0, jnp.float32)
    neg_ones = jnp.full((_LANES,), -1.0, jnp.float32)

    for r in range(_GRP):
        for s in range(_VPAD // 128):
            @plsc.parallel_loop(0, 128, step=_LANES, unroll=8)
            def _zero(j, r=r, s=s):
                c_v[r, s, pl.ds(j, _LANES)] = jnp.zeros((_LANES,), jnp.float32)

    def _scatter_pass(val):
        @plsc.parallel_loop(0, _L, unroll=8)
        def _step(l):
            xv = plsc.load_gather(x_v, [lane, jnp.full((_LANES,), l)])
            plsc.addupdate_scatter(
                c_v, [lane, jnp.right_shift(xv, 7), jnp.bitwise_and(xv, 127)],
                val)

    def _group(g_local, _):
        g = wid * _GPW + g_local
        base = g * _GRP
        pltpu.sync_copy(x_hbm.at[pl.ds(base, _GRP), :], x_v)
        _scatter_pass(ones)
        pltpu.sync_copy(c_v, c_hbm.at[pl.ds(base, _GRP), :, :])
        _scatter_pass(neg_ones)
        return _
    lax.fori_loop(0, _GPW, _group, None)


@functools.cache
def _make_count():
    return pl.kernel(
        _count_body,
        out_type=jax.ShapeDtypeStruct((_B, _VPAD // 128, 128), jnp.float32),
        mesh=plsc.VectorSubcoreMesh(core_axis_name="c", subcore_axis_name="s"),
        scratch_types=[
            pltpu.VMEM((_GRP, _L), jnp.int32),
            pltpu.VMEM((_GRP, _VPAD // 128, 128), jnp.float32),
        ],
        compiler_params=pltpu.CompilerParams(
            needs_layout_passes=False, use_tc_tiling_on_sc=False),
    )


_BLK = 1024


def _proj_body(c_ref, t_ref, w_ref, b_ref, o_ref):
    pooled = jnp.zeros((_BLK, _DIM), jnp.float32)
    for s in range(_VPAD // 128):
        pooled += jnp.dot(c_ref[:, s, :], t_ref[pl.ds(s * 128, 128), :],
                          preferred_element_type=jnp.float32)
    pooled = pooled * (1.0 / _L)
    logits_t = lax.dot_general(
        w_ref[...], pooled, (((1,), (1,)), ((), ())),
        preferred_element_type=jnp.float32)
    o_ref[...] = logits_t + b_ref[...]


_proj = pl.pallas_call(
    _proj_body,
    grid=(_B // _BLK,),
    in_specs=[
        pl.BlockSpec((_BLK, _VPAD // 128, 128), lambda i: (i, 0, 0)),
        pl.BlockSpec((_VPAD, _DIM), lambda i: (0, 0)),
        pl.BlockSpec((_VOCAB, _DIM), lambda i: (0, 0)),
        pl.BlockSpec((_VOCAB, 1), lambda i: (0, 0)),
    ],
    out_specs=pl.BlockSpec((_VOCAB, _BLK), lambda i: (0, i)),
    out_shape=jax.ShapeDtypeStruct((_VOCAB, _B), jnp.float32),
    compiler_params=pltpu.CompilerParams(
        dimension_semantics=("arbitrary",)),
)


def kernel(x, table, W, b):
    C = _make_count()(x.astype(jnp.int32))
    table_p = jnp.pad(table, ((0, _VPAD - _VOCAB), (0, 0)))
    return _proj(C, table_p, W, b.reshape(_VOCAB, 1)).T


# speedup vs baseline: 104.5102x; 1.4158x over previous
import functools

import jax
import jax.numpy as jnp
from jax import lax
from jax.experimental import pallas as pl
from jax.experimental.pallas import tpu as pltpu
from jax.experimental.pallas import tpu_sc as plsc

_VOCAB = 1001
_DIM = 64
_B = 16384
_L = 200
_VPAD = 1024

_NC, _NS, _LANES = 2, 16, 16
_NW = _NC * _NS
_GRP = _LANES
_NGROUPS = _B // _GRP
_GPW = _NGROUPS // _NW


def _count_body(x_hbm, c_hbm, x_v, c_v):
    wid = lax.axis_index("s") * _NC + lax.axis_index("c")
    lane = lax.iota(jnp.int32, _LANES)
    ones = jnp.full((_LANES,), 1.0, jnp.float32)
    neg_ones = jnp.full((_LANES,), -1.0, jnp.float32)
    tiles_pw = _GPW * _GRP // 128

    for r in range(_GRP):
        for s in range(_VPAD // 128):
            @plsc.parallel_loop(0, 128, step=_LANES, unroll=8)
            def _zero(j, r=r, s=s):
                c_v[r, s, pl.ds(j, _LANES)] = jnp.zeros((_LANES,), jnp.float32)

    pltpu.sync_copy(x_hbm.at[:, pl.ds(wid * tiles_pw, tiles_pw), :, :], x_v)

    def _scatter_pass(g_local, val):
        tile_l = lax.div(g_local, 8)
        c0 = lax.rem(g_local, 8) * _LANES

        @plsc.parallel_loop(0, _L, unroll=8)
        def _step(l):
            xv = x_v[jnp.right_shift(l, 3), tile_l,
                     jnp.bitwise_and(l, 7), pl.ds(c0, _LANES)]
            plsc.addupdate_scatter(
                c_v, [lane, jnp.right_shift(xv, 7), jnp.bitwise_and(xv, 127)],
                val)

    def _group(g_local, _):
        base = (wid * _GPW + g_local) * _GRP
        _scatter_pass(g_local, ones)
        pltpu.sync_copy(c_v, c_hbm.at[pl.ds(base, _GRP), :, :])
        _scatter_pass(g_local, neg_ones)
        return _
    lax.fori_loop(0, _GPW, _group, None)


@functools.cache
def _make_count():
    return pl.kernel(
        _count_body,
        out_type=jax.ShapeDtypeStruct((_B, _VPAD // 128, 128), jnp.float32),
        mesh=plsc.VectorSubcoreMesh(core_axis_name="c", subcore_axis_name="s"),
        scratch_types=[
            pltpu.VMEM((_L // 8, _GPW * _GRP // 128, 8, 128), jnp.int32),
            pltpu.VMEM((_GRP, _VPAD // 128, 128), jnp.float32),
        ],
        compiler_params=pltpu.CompilerParams(
            needs_layout_passes=False, use_tc_tiling_on_sc=False),
    )


_BLK = 1024


def _proj_body(c_ref, t_ref, w_ref, b_ref, o_ref):
    pooled = jnp.zeros((_BLK, _DIM), jnp.float32)
    for s in range(_VPAD // 128):
        pooled += jnp.dot(c_ref[:, s, :], t_ref[pl.ds(s * 128, 128), :],
                          preferred_element_type=jnp.float32)
    pooled = pooled * (1.0 / _L)
    logits_t = lax.dot_general(
        w_ref[...], pooled, (((1,), (1,)), ((), ())),
        preferred_element_type=jnp.float32)
    o_ref[...] = logits_t + b_ref[...]


_proj = pl.pallas_call(
    _proj_body,
    grid=(_B // _BLK,),
    in_specs=[
        pl.BlockSpec((_BLK, _VPAD // 128, 128), lambda i: (i, 0, 0)),
        pl.BlockSpec((_VPAD, _DIM), lambda i: (0, 0)),
        pl.BlockSpec((_VOCAB, _DIM), lambda i: (0, 0)),
        pl.BlockSpec((_VOCAB, 1), lambda i: (0, 0)),
    ],
    out_specs=pl.BlockSpec((_VOCAB, _BLK), lambda i: (0, i)),
    out_shape=jax.ShapeDtypeStruct((_VOCAB, _B), jnp.float32),
    compiler_params=pltpu.CompilerParams(
        dimension_semantics=("arbitrary",)),
)


def kernel(x, table, W, b):
    x4 = (x.astype(jnp.int32).T
          .reshape(_L // 8, 8, _B // 128, 128).transpose(0, 2, 1, 3))
    C = _make_count()(x4)
    table_p = jnp.pad(table, ((0, _VPAD - _VOCAB), (0, 0)))
    return _proj(C, table_p, W, b.reshape(_VOCAB, 1)).T


# speedup vs baseline: 117.1809x; 1.1212x over previous
import functools

import jax
import jax.numpy as jnp
from jax import lax
from jax.experimental import pallas as pl
from jax.experimental.pallas import tpu as pltpu
from jax.experimental.pallas import tpu_sc as plsc

_VOCAB = 1001
_DIM = 64
_B = 16384
_L = 200
_VPAD = 1024

_NC, _NS, _LANES = 2, 16, 16
_NW = _NC * _NS
_GRP = _LANES
_NCHUNK = 2
_BC = _B // _NCHUNK
_NGROUPS = _BC // _GRP
_GPW = _NGROUPS // _NW


def _count_body(x_hbm, c_hbm, x_v, c_v, *, chunk):
    wid = lax.axis_index("s") * _NC + lax.axis_index("c")
    lane = lax.iota(jnp.int32, _LANES)
    ones = jnp.full((_LANES,), 1.0, jnp.float32)
    neg_ones = jnp.full((_LANES,), -1.0, jnp.float32)
    tiles_pw = _GPW * _GRP // 128

    for r in range(_GRP):
        for s in range(_VPAD // 128):
            @plsc.parallel_loop(0, 128, step=_LANES, unroll=8)
            def _zero(j, r=r, s=s):
                c_v[r, s, pl.ds(j, _LANES)] = jnp.zeros((_LANES,), jnp.float32)

    x_tile0 = chunk * (_BC // 128) + wid * tiles_pw
    pltpu.sync_copy(x_hbm.at[:, pl.ds(x_tile0, tiles_pw), :, :], x_v)

    def _scatter_pass(g_local, val):
        tile_l = lax.div(g_local, 8)
        c0 = lax.rem(g_local, 8) * _LANES

        @plsc.parallel_loop(0, _L, unroll=8)
        def _step(l):
            xv = x_v[jnp.right_shift(l, 3), tile_l,
                     jnp.bitwise_and(l, 7), pl.ds(c0, _LANES)]
            plsc.addupdate_scatter(
                c_v, [lane, jnp.right_shift(xv, 7), jnp.bitwise_and(xv, 127)],
                val)

    def _group(g_local, _):
        base = (wid * _GPW + g_local) * _GRP
        _scatter_pass(g_local, ones)
        pltpu.sync_copy(c_v, c_hbm.at[pl.ds(base, _GRP), :, :])
        _scatter_pass(g_local, neg_ones)
        return _
    lax.fori_loop(0, _GPW, _group, None)


@functools.cache
def _make_count(chunk):
    return pl.kernel(
        functools.partial(_count_body, chunk=chunk),
        out_type=jax.ShapeDtypeStruct((_BC, _VPAD // 128, 128), jnp.float32),
        mesh=plsc.VectorSubcoreMesh(core_axis_name="c", subcore_axis_name="s"),
        scratch_types=[
            pltpu.VMEM((_L // 8, _GPW * _GRP // 128, 8, 128), jnp.int32),
            pltpu.VMEM((_GRP, _VPAD // 128, 128), jnp.float32),
        ],
        compiler_params=pltpu.CompilerParams(
            needs_layout_passes=False, use_tc_tiling_on_sc=False),
    )


_BLK = 1024


def _proj_body(c_ref, t_ref, w_ref, b_ref, o_ref):
    pooled = jnp.zeros((_BLK, _DIM), jnp.float32)
    for s in range(_VPAD // 128):
        pooled += jnp.dot(c_ref[:, s, :], t_ref[pl.ds(s * 128, 128), :],
                          preferred_element_type=jnp.float32)
    pooled = pooled * (1.0 / _L)
    logits_t = lax.dot_general(
        w_ref[...], pooled, (((1,), (1,)), ((), ())),
        preferred_element_type=jnp.float32)
    o_ref[...] = logits_t + b_ref[...]


def _proj_body_aliased(prev_ref, c_ref, t_ref, w_ref, b_ref, o_ref):
    del prev_ref
    _proj_body(c_ref, t_ref, w_ref, b_ref, o_ref)


_proj0 = pl.pallas_call(
    _proj_body,
    grid=(_BC // _BLK,),
    in_specs=[
        pl.BlockSpec((_BLK, _VPAD // 128, 128), lambda i: (i, 0, 0)),
        pl.BlockSpec((_VPAD, _DIM), lambda i: (0, 0)),
        pl.BlockSpec((_VOCAB, _DIM), lambda i: (0, 0)),
        pl.BlockSpec((_VOCAB, 1), lambda i: (0, 0)),
    ],
    out_specs=pl.BlockSpec((_VOCAB, _BLK), lambda i: (0, i)),
    out_shape=jax.ShapeDtypeStruct((_VOCAB, _B), jnp.float32),
    compiler_params=pltpu.CompilerParams(
        dimension_semantics=("arbitrary",)),
)

_proj1 = pl.pallas_call(
    _proj_body_aliased,
    grid=(_BC // _BLK,),
    in_specs=[
        pl.BlockSpec(memory_space=pl.ANY),
        pl.BlockSpec((_BLK, _VPAD // 128, 128), lambda i: (i, 0, 0)),
        pl.BlockSpec((_VPAD, _DIM), lambda i: (0, 0)),
        pl.BlockSpec((_VOCAB, _DIM), lambda i: (0, 0)),
        pl.BlockSpec((_VOCAB, 1), lambda i: (0, 0)),
    ],
    out_specs=pl.BlockSpec((_VOCAB, _BLK),
                           lambda i: (0, i + _BC // _BLK)),
    out_shape=jax.ShapeDtypeStruct((_VOCAB, _B), jnp.float32),
    input_output_aliases={0: 0},
    compiler_params=pltpu.CompilerParams(
        dimension_semantics=("arbitrary",)),
)


def kernel(x, table, W, b):
    x4 = (x.astype(jnp.int32).T
          .reshape(_L // 8, 8, _B // 128, 128).transpose(0, 2, 1, 3))
    table_p = jnp.pad(table, ((0, _VPAD - _VOCAB), (0, 0)))
    bcol = b.reshape(_VOCAB, 1)
    C0 = _make_count(0)(x4)
    logits_t = _proj0(C0, table_p, W, bcol)
    C1 = _make_count(1)(x4)
    logits_t = _proj1(logits_t, C1, table_p, W, bcol)
    return logits_t.T
